# baseline (speedup 1.0000x reference)
import jax
import jax.numpy as jnp
import numpy as np
from jax.experimental import pallas as pl
from jax.experimental.pallas import tpu as pltpu

_DIM = 128
_HEADS = 8
_HD = _DIM // _HEADS


def _fused_body(x_ref, wmu_ref, bmu_ref, inw_ref, inb_ref, ow_ref, ob_ref,
                wo_ref, bo_ref, out_ref):
    f32 = jnp.float32
    xb = x_ref[0]
    w1 = wmu_ref[:, :_DIM]
    enh = jax.lax.dot_general(xb, w1, (((1,), (1,)), ((), ())),
                              preferred_element_type=f32) + bmu_ref[:]
    qkv = jax.lax.dot_general(enh, inw_ref[:], (((1,), (1,)), ((), ())),
                              preferred_element_type=f32) + inb_ref[:]
    scale = np.float32(np.log2(np.e) / np.sqrt(_HD))
    bf16 = jnp.bfloat16
    qs = (qkv[:, :_DIM] * scale).astype(bf16)
    ks = qkv[:, _DIM:2 * _DIM].astype(bf16)
    outs = []
    for h in range(_HEADS):
        lo = h * _HD
        q = qs[:, lo:lo + _HD]
        k = ks[:, lo:lo + _HD]
        v = qkv[:, 2 * _DIM + lo:2 * _DIM + lo + _HD].astype(bf16)
        s = jax.lax.dot_general(q, k, (((1,), (1,)), ((), ())),
                                preferred_element_type=f32)
        m = jnp.max(s, axis=1, keepdims=True)
        e = jnp.exp2(s - m)
        r = 1.0 / jnp.sum(e, axis=1, keepdims=True)
        pv = jax.lax.dot_general(e.astype(bf16), v, (((1,), (0,)), ((), ())),
                                 preferred_element_type=f32)
        outs.append(pv * r)
    o = jnp.concatenate(outs, axis=1)
    att = jax.lax.dot_general(o, ow_ref[:], (((1,), (1,)), ((), ())),
                              preferred_element_type=f32) + ob_ref[:]
    y = jax.lax.dot_general(att, wo_ref[:], (((1,), (1,)), ((), ())),
                            preferred_element_type=f32) + bo_ref[:]
    out_ref[0] = y


def kernel(x, support_examples, support_labels, memory_bank, memory_usage,
           memory_labels, W_mu, b_mu, in_proj_w, in_proj_b, attn_out_w,
           attn_out_b, W_out, b_out):
    B, S, D = x.shape

    def full(shape):
        return pl.BlockSpec(shape, lambda b: (0,) * len(shape))

    return pl.pallas_call(
        _fused_body,
        grid=(B,),
        in_specs=[
            pl.BlockSpec((1, S, D), lambda b: (b, 0, 0)),
            full(W_mu.shape),
            full((1, D)),
            full(in_proj_w.shape),
            full((1, 3 * D)),
            full(attn_out_w.shape),
            full((1, D)),
            full(W_out.shape),
            full((1, D)),
        ],
        out_specs=pl.BlockSpec((1, S, D), lambda b: (b, 0, 0)),
        out_shape=jax.ShapeDtypeStruct((B, S, D), x.dtype),
        compiler_params=pltpu.CompilerParams(
            dimension_semantics=("parallel",)),
    )(x, W_mu, b_mu.reshape(1, -1), in_proj_w, in_proj_b.reshape(1, -1),
      attn_out_w, attn_out_b.reshape(1, -1), W_out, b_out.reshape(1, -1))


# speedup vs baseline: 1.6168x; 1.6168x over previous
import jax
import jax.numpy as jnp
import numpy as np
from jax.experimental import pallas as pl
from jax.experimental.pallas import tpu as pltpu

_DIM = 128
_HEADS = 8
_HD = _DIM // _HEADS


def _fused_body(x_ref, wmu_ref, bmu_ref, inw_ref, inb_ref, ow_ref, ob_ref,
                wo_ref, bo_ref, out_ref):
    f32 = jnp.float32
    xb = x_ref[0]
    w1 = wmu_ref[:, :_DIM]
    enh = jax.lax.dot_general(xb, w1, (((1,), (1,)), ((), ())),
                              preferred_element_type=f32) + bmu_ref[:]
    qkv = jax.lax.dot_general(enh, inw_ref[:], (((1,), (1,)), ((), ())),
                              preferred_element_type=f32) + inb_ref[:]
    scale = np.float32(np.log2(np.e) / np.sqrt(_HD))
    bf16 = jnp.bfloat16
    qs = (qkv[:, :_DIM] * scale).astype(bf16)
    ks = qkv[:, _DIM:2 * _DIM].astype(bf16)
    vs = qkv[:, 2 * _DIM:3 * _DIM].astype(bf16)
    H = range(_HEADS)
    sl = [slice(h * _HD, (h + 1) * _HD) for h in H]
    ss = [jax.lax.dot_general(qs[:, sl[h]], ks[:, sl[h]],
                              (((1,), (1,)), ((), ())),
                              preferred_element_type=f32) for h in H]
    ms = [jnp.max(s, axis=1, keepdims=True) for s in ss]
    es = [jnp.exp2(s - m) for s, m in zip(ss, ms)]
    rs = [1.0 / jnp.sum(e, axis=1, keepdims=True) for e in es]
    pvs = [jax.lax.dot_general(e.astype(bf16), vs[:, sl[h]],
                               (((1,), (0,)), ((), ())),
                               preferred_element_type=f32)
           for h, e in zip(H, es)]
    outs = [pv * r for pv, r in zip(pvs, rs)]
    o = jnp.concatenate(outs, axis=1)
    att = jax.lax.dot_general(o, ow_ref[:], (((1,), (1,)), ((), ())),
                              preferred_element_type=f32) + ob_ref[:]
    y = jax.lax.dot_general(att, wo_ref[:], (((1,), (1,)), ((), ())),
                            preferred_element_type=f32) + bo_ref[:]
    out_ref[0] = y


def kernel(x, support_examples, support_labels, memory_bank, memory_usage,
           memory_labels, W_mu, b_mu, in_proj_w, in_proj_b, attn_out_w,
           attn_out_b, W_out, b_out):
    B, S, D = x.shape

    def full(shape):
        return pl.BlockSpec(shape, lambda b: (0,) * len(shape))

    return pl.pallas_call(
        _fused_body,
        grid=(B,),
        in_specs=[
            pl.BlockSpec((1, S, D), lambda b: (b, 0, 0)),
            full(W_mu.shape),
            full((1, D)),
            full(in_proj_w.shape),
            full((1, 3 * D)),
            full(attn_out_w.shape),
            full((1, D)),
            full(W_out.shape),
            full((1, D)),
        ],
        out_specs=pl.BlockSpec((1, S, D), lambda b: (b, 0, 0)),
        out_shape=jax.ShapeDtypeStruct((B, S, D), x.dtype),
        compiler_params=pltpu.CompilerParams(
            dimension_semantics=("parallel",)),
    )(x, W_mu, b_mu.reshape(1, -1), in_proj_w, in_proj_b.reshape(1, -1),
      attn_out_w, attn_out_b.reshape(1, -1), W_out, b_out.reshape(1, -1))


# speedup vs baseline: 2.0036x; 1.2392x over previous
import jax
import jax.numpy as jnp
import numpy as np
from jax.experimental import pallas as pl
from jax.experimental.pallas import tpu as pltpu

_DIM = 128
_HEADS = 8
_HD = _DIM // _HEADS


def _fused_body(x_ref, wmu_ref, bmu_ref, inw_ref, inb_ref, ow_ref, ob_ref,
                wo_ref, bo_ref, out_ref):
    f32 = jnp.float32
    xb = x_ref[0]
    w1 = wmu_ref[:, :_DIM]
    enh = jax.lax.dot_general(xb, w1, (((1,), (1,)), ((), ())),
                              preferred_element_type=f32) + bmu_ref[:]
    qkv = (jax.lax.dot_general(enh, inw_ref[:], (((1,), (1,)), ((), ())),
                               preferred_element_type=f32)
           + inb_ref[:]).astype(jnp.bfloat16)
    bf16 = jnp.bfloat16
    S = qkv.shape[0]
    ones = jnp.ones((S, _HD), bf16)
    H = range(_HEADS)
    sl = [slice(h * _HD, (h + 1) * _HD) for h in H]
    ss = [jax.lax.dot_general(qkv[:, sl[h]], qkv[:, _DIM:][:, sl[h]],
                              (((1,), (1,)), ((), ())),
                              preferred_element_type=f32) for h in H]
    ms = [jnp.max(jnp.max(s, axis=0, keepdims=True), axis=1, keepdims=True)
          for s in ss]
    es = [jnp.exp2((s - m).astype(bf16)) for s, m in zip(ss, ms)]
    vaugs = [jnp.concatenate([qkv[:, 2 * _DIM:][:, sl[h]], ones], axis=1)
             for h in H]
    pvs = [jax.lax.dot_general(e, va, (((1,), (0,)), ((), ())),
                               preferred_element_type=f32)
           for e, va in zip(es, vaugs)]
    outs = [pv[:, :_HD] / pv[:, _HD:2 * _HD] for pv in pvs]
    o = jnp.concatenate(outs, axis=1)
    att = jax.lax.dot_general(o, ow_ref[:], (((1,), (1,)), ((), ())),
                              preferred_element_type=f32) + ob_ref[:]
    y = jax.lax.dot_general(att, wo_ref[:], (((1,), (1,)), ((), ())),
                            preferred_element_type=f32) + bo_ref[:]
    out_ref[0] = y


def kernel(x, support_examples, support_labels, memory_bank, memory_usage,
           memory_labels, W_mu, b_mu, in_proj_w, in_proj_b, attn_out_w,
           attn_out_b, W_out, b_out):
    B, S, D = x.shape
    qscale = np.float32(np.log2(np.e) / np.sqrt(_HD))
    row_scale = jnp.concatenate(
        [jnp.full((D,), qscale, jnp.float32), jnp.ones((2 * D,), jnp.float32)])
    in_proj_w = in_proj_w * row_scale[:, None]
    in_proj_b = in_proj_b * row_scale

    def full(shape):
        return pl.BlockSpec(shape, lambda b: (0,) * len(shape))

    return pl.pallas_call(
        _fused_body,
        grid=(B,),
        in_specs=[
            pl.BlockSpec((1, S, D), lambda b: (b, 0, 0)),
            full(W_mu.shape),
            full((1, D)),
            full(in_proj_w.shape),
            full((1, 3 * D)),
            full(attn_out_w.shape),
            full((1, D)),
            full(W_out.shape),
            full((1, D)),
        ],
        out_specs=pl.BlockSpec((1, S, D), lambda b: (b, 0, 0)),
        out_shape=jax.ShapeDtypeStruct((B, S, D), x.dtype),
        compiler_params=pltpu.CompilerParams(
            dimension_semantics=("parallel",)),
    )(x, W_mu, b_mu.reshape(1, -1), in_proj_w, in_proj_b.reshape(1, -1),
      attn_out_w, attn_out_b.reshape(1, -1), W_out, b_out.reshape(1, -1))
